# Initial kernel scaffold; baseline (speedup 1.0000x reference)
import jax
import jax.numpy as jnp
from jax.experimental import pallas as pl


def kernel(z_e, codebook):
    raise NotImplementedError("write your pallas kernel here")


# speedup vs baseline: 1.6167x; 1.6167x over previous
import functools

import jax
import jax.numpy as jnp
from jax import lax
from jax.experimental import pallas as pl
from jax.experimental.pallas import tpu as pltpu
from jax.experimental.pallas import tpu_sc as plsc


ROW_BLK = 512


def _dist_argmin_body(z_ref, cb_ref, codes_ref, c2_ref):
    i = pl.program_id(0)

    @pl.when(i == 0)
    def _():
        cb = cb_ref[...]
        c2_ref[...] = jnp.sum(cb * cb, axis=1)[None, :]

    z = z_ref[...]
    zb = (2.0 * z).astype(jnp.bfloat16)
    s = lax.dot_general(
        zb, cb_ref[...], (((1,), (1,)), ((), ())),
        preferred_element_type=jnp.float32)
    z2 = jnp.sum(z * z, axis=1, keepdims=True)
    d = (z2 - s) + c2_ref[...]
    codes_ref[0, 0, :] = jnp.argmin(d, axis=1).astype(jnp.int32)


def _dist_argmin(z_flat, codebook):
    n, dmodel = z_flat.shape
    k = codebook.shape[0]
    nblk = n // ROW_BLK
    codes3 = pl.pallas_call(
        _dist_argmin_body,
        grid=(nblk,),
        in_specs=[
            pl.BlockSpec((ROW_BLK, dmodel), lambda i: (i, 0)),
            pl.BlockSpec((k, dmodel), lambda i: (0, 0)),
        ],
        out_specs=pl.BlockSpec((1, 1, ROW_BLK), lambda i: (i, 0, 0)),
        out_shape=jax.ShapeDtypeStruct((nblk, 1, ROW_BLK), jnp.int32),
        scratch_shapes=[pltpu.VMEM((1, k), jnp.float32)],
    )(z_flat, codebook)
    return codes3.reshape(n)


def _sc_gather(codebook, codes, n_rows):
    dmodel = codebook.shape[1]
    info = plsc.get_sparse_core_info()
    nc, ns = info.num_cores, info.num_subcores
    nw = nc * ns
    chunk = 128
    rows_per_w = n_rows // nw
    nchunk = rows_per_w // chunk
    idx3 = codes.reshape(nw, nchunk, chunk)
    mesh = plsc.VectorSubcoreMesh(core_axis_name="c", subcore_axis_name="s")

    @functools.partial(
        pl.kernel,
        out_type=jax.ShapeDtypeStruct((n_rows, dmodel), jnp.float32),
        mesh=mesh,
        scratch_types=[
            pltpu.VMEM((nchunk, chunk), jnp.int32),
            pltpu.VMEM((chunk, dmodel), jnp.float32),
            pltpu.SemaphoreType.DMA,
        ],
    )
    def gather(table_hbm, idx_hbm, out_hbm, idx_v, rows_v, sem):
        wid = lax.axis_index("s") * nc + lax.axis_index("c")
        base = wid * rows_per_w
        pltpu.sync_copy(idx_hbm.at[wid], idx_v)
        for j in range(nchunk):
            pltpu.async_copy(table_hbm.at[idx_v.at[j]], rows_v, sem).wait()
            pltpu.sync_copy(rows_v, out_hbm.at[pl.ds(base + j * chunk, chunk)])

    return gather(codebook, idx3)


def _finalize_body(z_ref, zq_ref, out_ref, loss_ref):
    i = pl.program_id(0)

    @pl.when(i == 0)
    def _():
        loss_ref[0, 0] = 0.0

    z = z_ref[...]
    q = zq_ref[...]
    out_ref[...] = z + (q - z)
    d = z - q
    loss_ref[0, 0] += jnp.sum(d * d)


def _finalize(z_flat, zq_flat):
    n, dmodel = z_flat.shape
    nblk = n // ROW_BLK
    zqst, loss = pl.pallas_call(
        _finalize_body,
        grid=(nblk,),
        in_specs=[
            pl.BlockSpec((ROW_BLK, dmodel), lambda i: (i, 0)),
            pl.BlockSpec((ROW_BLK, dmodel), lambda i: (i, 0)),
        ],
        out_specs=[
            pl.BlockSpec((ROW_BLK, dmodel), lambda i: (i, 0)),
            pl.BlockSpec(memory_space=pltpu.SMEM),
        ],
        out_shape=[
            jax.ShapeDtypeStruct((n, dmodel), jnp.float32),
            jax.ShapeDtypeStruct((1, 1), jnp.float32),
        ],
    )(z_flat, zq_flat)
    scale = 1.25 / (n * dmodel)
    return zqst, (loss[0, 0] * scale).astype(jnp.float32)


def kernel(z_e, codebook):
    b, dmodel, h, w = z_e.shape
    hw = h * w
    n = b * hw
    z_flat = z_e.reshape(b, dmodel, hw).transpose(0, 2, 1).reshape(n, dmodel)
    codes = _dist_argmin(z_flat, codebook)
    zq_flat = _sc_gather(codebook, codes, n)
    zqst_flat, loss = _finalize(z_flat, zq_flat)
    z_q_st = zqst_flat.reshape(b, hw, dmodel).transpose(0, 2, 1).reshape(z_e.shape)
    return z_q_st, loss, codes.reshape(b, hw)
